# baseline (speedup 1.0000x reference)
import functools

import jax
import jax.numpy as jnp
from jax.experimental import pallas as pl
from jax.experimental.pallas import tpu as pltpu

_B, _T, _D_IN, _G, _D = 8, 2048, 512, 8, 256
_N = _B * _T
_BLK = 512


def _fused_body(x_ref, wpi_ref, bpi_ref, wsig_ref, bsig_ref, wmu_ref, bmu_ref,
                mu_ref, sig_ref):
    x = x_ref[...]
    logits = jnp.dot(x, wpi_ref[...], preferred_element_type=jnp.float32)
    logits = logits + bpi_ref[...]
    g = jnp.argmax(logits, axis=1).astype(jnp.int32)[:, None]

    xh = x.astype(jnp.bfloat16)
    mu_full = jnp.dot(xh, wmu_ref[...], preferred_element_type=jnp.float32)
    sig_full = jnp.dot(xh, wsig_ref[...], preferred_element_type=jnp.float32)

    acc_mu = mu_full[:, :_D] + bmu_ref[:, :_D]
    acc_sig = sig_full[:, :_D] + bsig_ref[:, :_D]
    for k in range(1, _G):
        sel = g == k
        acc_mu = jnp.where(sel, mu_full[:, k * _D:(k + 1) * _D]
                           + bmu_ref[:, k * _D:(k + 1) * _D], acc_mu)
        acc_sig = jnp.where(sel, sig_full[:, k * _D:(k + 1) * _D]
                            + bsig_ref[:, k * _D:(k + 1) * _D], acc_sig)
    mu_ref[...] = acc_mu
    sig_ref[...] = jnp.exp(acc_sig)


@jax.jit
def kernel(x, W_pi, b_pi, W_sigma, b_sigma, W_mu, b_mu):
    xf = x.reshape(_N, _D_IN)
    wsig_h = W_sigma.astype(jnp.bfloat16)
    wmu_h = W_mu.astype(jnp.bfloat16)
    grid = (_N // _BLK,)
    full = lambda i: (0, 0)
    mu, sig = pl.pallas_call(
        _fused_body,
        grid=grid,
        in_specs=[
            pl.BlockSpec((_BLK, _D_IN), lambda i: (i, 0)),
            pl.BlockSpec((_D_IN, _G), full),
            pl.BlockSpec((_G,), lambda i: (0,)),
            pl.BlockSpec((_D_IN, _G * _D), full),
            pl.BlockSpec((1, _G * _D), full),
            pl.BlockSpec((_D_IN, _G * _D), full),
            pl.BlockSpec((1, _G * _D), full),
        ],
        out_specs=[
            pl.BlockSpec((_BLK, _D), lambda i: (i, 0)),
            pl.BlockSpec((_BLK, _D), lambda i: (i, 0)),
        ],
        out_shape=[
            jax.ShapeDtypeStruct((_N, _D), jnp.float32),
            jax.ShapeDtypeStruct((_N, _D), jnp.float32),
        ],
        compiler_params=pltpu.CompilerParams(
            dimension_semantics=("arbitrary",),
        ),
    )(xf, W_pi, b_pi, wsig_h, b_sigma[None, :], wmu_h, b_mu[None, :])
    return mu.reshape(_B, _T, _D), sig.reshape(_B, _T, _D)
